# Initial kernel scaffold; baseline (speedup 1.0000x reference)
import jax
import jax.numpy as jnp
from jax.experimental import pallas as pl


def kernel(seq_lens, last_loc, free_page):
    raise NotImplementedError("write your pallas kernel here")


# speedup vs baseline: 1.7224x; 1.7224x over previous
import jax
import jax.numpy as jnp
from jax import lax
from jax.experimental import pallas as pl
from jax.experimental.pallas import tpu as pltpu
from jax.experimental.pallas import tpu_sc as plsc

B = 65536
L = 16
NC = 2
NS = 16
NW = NC * NS
STRIPE = B // NS
BLK = B // NW
CH1 = STRIPE // L
CH2 = BLK // L
FBUF = BLK + L


def _nnp(sv):
    return jnp.right_shift(sv + 15, 4) - jnp.right_shift(sv + 14, 4)


def _body(seq_hbm, last_hbm, free_hbm, out_hbm,
          seq1_v, seq2_v, last_v, free_v, out_v, stage_v, tots_v, tot_hbm):
    c = lax.axis_index("c")
    s = lax.axis_index("s")

    pltpu.sync_copy(seq_hbm.at[pl.ds(s * STRIPE, STRIPE)], seq1_v)

    for b in range(2):
        def p1(i, acc, b=b):
            sv = seq1_v[pl.ds(b * BLK + i * L, L)]
            return acc + _nnp(sv)
        acc = lax.fori_loop(0, CH2, p1, jnp.zeros((L,), jnp.int32))
        stage_v[b] = jnp.full((L,), jnp.sum(acc), jnp.int32)

    pltpu.sync_copy(stage_v, tot_hbm.at[c].at[pl.ds(2 * s, 2)])
    plsc.subcore_barrier()
    pltpu.sync_copy(tot_hbm.at[c], tots_v)

    k = c * NS + s
    off = jnp.zeros((L,), jnp.int32)
    for j in range(NW):
        off = jnp.where(j < k, off + tots_v[j], off)
    off_s = jnp.max(off)

    base = k * BLK
    pltpu.sync_copy(seq_hbm.at[pl.ds(base, BLK)], seq2_v)
    pltpu.sync_copy(last_hbm.at[pl.ds(base, BLK)], last_v)
    start = pl.multiple_of(
        jnp.minimum(jnp.bitwise_and(off_s, -L), B - FBUF), L)
    pltpu.sync_copy(free_hbm.at[pl.ds(start, FBUF)], free_v)
    adj = off_s - start

    def p2(i, carry):
        sv = seq2_v[pl.ds(i * L, L)]
        nnp = _nnp(sv)
        inc = plsc.cumsum(nnp)
        excl = inc - nnp + carry
        idx = jnp.minimum(excl + adj, FBUF - 1)
        page = plsc.load_gather(free_v, [idx]) * L
        ll = last_v[pl.ds(i * L, L)]
        out_v[pl.ds(i * L, L)] = jnp.where(nnp != 0, page, ll + 1)
        return carry + jnp.sum(nnp)
    lax.fori_loop(0, CH2, p2, jnp.int32(0))

    pltpu.sync_copy(out_v, out_hbm.at[pl.ds(base, BLK)])


def kernel(seq_lens, last_loc, free_page):
    run = pl.kernel(
        _body,
        out_type=jax.ShapeDtypeStruct((B,), jnp.int32),
        mesh=plsc.VectorSubcoreMesh(core_axis_name="c", subcore_axis_name="s"),
        compiler_params=pltpu.CompilerParams(needs_layout_passes=False),
        scratch_types=[
            pltpu.VMEM((STRIPE,), jnp.int32),
            pltpu.VMEM((BLK,), jnp.int32),
            pltpu.VMEM((BLK,), jnp.int32),
            pltpu.VMEM((FBUF,), jnp.int32),
            pltpu.VMEM((BLK,), jnp.int32),
            pltpu.VMEM((2, L), jnp.int32),
            pltpu.VMEM((NW, L), jnp.int32),
            pltpu.MemorySpace.HBM((NC, NW, L), jnp.int32),
        ],
    )
    return run(seq_lens.astype(jnp.int32),
               last_loc.astype(jnp.int32),
               free_page.astype(jnp.int32))


# speedup vs baseline: 1.8881x; 1.0962x over previous
import jax
import jax.numpy as jnp
from jax import lax
from jax.experimental import pallas as pl
from jax.experimental.pallas import tpu as pltpu
from jax.experimental.pallas import tpu_sc as plsc

B = 65536
L = 16
NC = 2
NS = 16
NW = NC * NS
STRIPE = B // NS
BLK = B // NW
CH2 = BLK // L
FBUF = BLK + L


def _nnp(sv):
    return jnp.right_shift(sv + 15, 4) - jnp.right_shift(sv + 14, 4)


def _body(seq_hbm, last_hbm, free_hbm, out_hbm,
          seq1_v, seq2_v, last_v, free_v, out_v, stage_v, tots_v, tot_hbm,
          sem_seq2, sem_last):
    c = lax.axis_index("c")
    s = lax.axis_index("s")
    k = c * NS + s
    base = k * BLK

    cp_seq2 = pltpu.async_copy(seq_hbm.at[pl.ds(base, BLK)], seq2_v, sem_seq2)
    cp_last = pltpu.async_copy(last_hbm.at[pl.ds(base, BLK)], last_v, sem_last)

    pltpu.sync_copy(seq_hbm.at[pl.ds(s * STRIPE, STRIPE)], seq1_v)

    for b in range(2):
        @plsc.parallel_loop(0, CH2, unroll=8,
                            carry=jnp.zeros((L,), jnp.int32))
        def p1(i, acc, b=b):
            sv = seq1_v[pl.ds(b * BLK + i * L, L)]
            return acc + _nnp(sv)
        stage_v[b] = jnp.full((L,), jnp.sum(p1), jnp.int32)

    pltpu.sync_copy(stage_v, tot_hbm.at[c].at[pl.ds(2 * s, 2)])
    plsc.subcore_barrier()
    pltpu.sync_copy(tot_hbm.at[c], tots_v)

    off = jnp.zeros((L,), jnp.int32)
    for j in range(NW):
        off = jnp.where(j < k, off + tots_v[j], off)
    off_s = jnp.max(off)

    start = pl.multiple_of(
        jnp.minimum(jnp.bitwise_and(off_s, -L), B - FBUF), L)
    pltpu.sync_copy(free_hbm.at[pl.ds(start, FBUF)], free_v)
    adj = off_s - start
    cp_seq2.wait()
    cp_last.wait()

    @plsc.parallel_loop(0, CH2, unroll=4, carry=jnp.int32(0))
    def p2(i, carry):
        sv = seq2_v[pl.ds(i * L, L)]
        nnp = _nnp(sv)
        inc = plsc.cumsum(nnp)
        excl = inc - nnp + carry
        idx = jnp.minimum(excl + adj, FBUF - 1)
        page = plsc.load_gather(free_v, [idx]) * L
        ll = last_v[pl.ds(i * L, L)]
        out_v[pl.ds(i * L, L)] = jnp.where(nnp != 0, page, ll + 1)
        return carry + jnp.sum(nnp)

    pltpu.sync_copy(out_v, out_hbm.at[pl.ds(base, BLK)])


def kernel(seq_lens, last_loc, free_page):
    run = pl.kernel(
        _body,
        out_type=jax.ShapeDtypeStruct((B,), jnp.int32),
        mesh=plsc.VectorSubcoreMesh(core_axis_name="c", subcore_axis_name="s"),
        compiler_params=pltpu.CompilerParams(needs_layout_passes=False),
        scratch_types=[
            pltpu.VMEM((STRIPE,), jnp.int32),
            pltpu.VMEM((BLK,), jnp.int32),
            pltpu.VMEM((BLK,), jnp.int32),
            pltpu.VMEM((FBUF,), jnp.int32),
            pltpu.VMEM((BLK,), jnp.int32),
            pltpu.VMEM((2, L), jnp.int32),
            pltpu.VMEM((NW, L), jnp.int32),
            pltpu.MemorySpace.HBM((NC, NW, L), jnp.int32),
            pltpu.SemaphoreType.DMA,
            pltpu.SemaphoreType.DMA,
        ],
    )
    return run(seq_lens.astype(jnp.int32),
               last_loc.astype(jnp.int32),
               free_page.astype(jnp.int32))
